# baseline (speedup 1.0000x reference)
import functools

import jax
import jax.numpy as jnp
from jax import lax
from jax.experimental import pallas as pl
from jax.experimental.pallas import tpu as pltpu
from jax.experimental.pallas import tpu_sc as plsc


def _sc_gather_t(x, embT):
    D, V = embT.shape
    (B,) = x.shape
    info = plsc.get_sparse_core_info()
    nc = info.num_cores
    nw = nc * info.num_subcores
    bh = B * D // nw

    mesh = plsc.VectorSubcoreMesh(core_axis_name="c", subcore_axis_name="s")

    @functools.partial(
        pl.kernel,
        mesh=mesh,
        out_type=jax.ShapeDtypeStruct((D, B), jnp.float32),
        compiler_params=pltpu.CompilerParams(
            use_tc_tiling_on_sc=True, needs_layout_passes=False
        ),
        scratch_types=[
            pltpu.VMEM((V,), jnp.float32),
            pltpu.VMEM((bh,), jnp.int32),
            pltpu.VMEM((bh,), jnp.float32),
            pltpu.SemaphoreType.DMA,
            pltpu.SemaphoreType.DMA,
        ],
    )
    def gather_k(idx_hbm, table_hbm, out_hbm, row_v, idx_v, h_v, sem_r, sem_i):
        wid = lax.axis_index("s") * nc + lax.axis_index("c")
        l = wid // 2
        half = wid % 2
        cp_row = pltpu.async_copy(table_hbm.at[l], row_v, sem_r)
        cp_idx = pltpu.async_copy(idx_hbm.at[pl.ds(half * bh, bh)], idx_v, sem_i)
        cp_idx.wait()
        cp_row.wait()
        for g in range(bh // 16):
            xv = idx_v[pl.ds(g * 16, 16)]
            h_v[pl.ds(g * 16, 16)] = plsc.load_gather(row_v, [xv])
        pltpu.sync_copy(h_v, out_hbm.at[l, pl.ds(half * bh, bh)])

    return gather_k(x, embT)


def _tc_project_t(hT, Wt, brow, vt, nbuf):
    D, B = hT.shape
    V = Wt.shape[1]
    nfull = V // vt
    rem = V - nfull * vt
    grid = nfull + (1 if rem else 0)

    def body(ht_ref, wt_ref, b_ref, out_hbm, bufs, sems):
        i = pl.program_id(0)
        n = pl.num_programs(0)
        slot = lax.rem(i, nbuf)

        def copy_for(step, s, width):
            return pltpu.make_async_copy(
                bufs.at[s, pl.ds(0, width), :],
                out_hbm.at[pl.ds(step * vt, width), :],
                sems.at[s],
            )

        @pl.when(i >= nbuf)
        def _():
            copy_for(i - nbuf, slot, vt).wait()

        val = lax.dot_general(
            wt_ref[...],
            ht_ref[...],
            dimension_numbers=(((0,), (0,)), ((), ())),
            preferred_element_type=jnp.float32,
        ) + jnp.transpose(b_ref[...], (1, 0))
        bufs[slot] = val

        if rem:
            @pl.when(i < nfull)
            def _():
                copy_for(i, slot, vt).start()

            @pl.when(i == nfull)
            def _():
                copy_for(nfull, slot, rem).start()
        else:
            copy_for(i, slot, vt).start()

        @pl.when(i == n - 1)
        def _():
            for k in range(min(nbuf, grid)):
                step = grid - 1 - k
                width = rem if (rem and step == nfull) else vt
                copy_for(step, step % nbuf, width).wait()

    return pl.pallas_call(
        body,
        grid=(grid,),
        compiler_params=pltpu.CompilerParams(
            vmem_limit_bytes=100 * 1024 * 1024
        ),
        in_specs=[
            pl.BlockSpec((D, B), lambda i: (0, 0)),
            pl.BlockSpec((D, vt), lambda i: (0, i)),
            pl.BlockSpec((1, vt), lambda i: (0, i)),
        ],
        out_specs=pl.BlockSpec(memory_space=pl.ANY),
        out_shape=jax.ShapeDtypeStruct((V, B), jnp.float32),
        scratch_shapes=[
            pltpu.VMEM((nbuf, vt, B), jnp.float32),
            pltpu.SemaphoreType.DMA((nbuf,)),
        ],
    )(hT, Wt, brow)


def kernel(x, emb, W, b):
    hT = _sc_gather_t(x.astype(jnp.int32), emb.T)
    out_t = _tc_project_t(hT, W.T, b.reshape(1, -1), vt=3200, nbuf=4)
    return out_t.T


# speedup vs baseline: 1.0290x; 1.0290x over previous
import functools

import jax
import jax.numpy as jnp
from jax import lax
from jax.experimental import pallas as pl
from jax.experimental.pallas import tpu as pltpu
from jax.experimental.pallas import tpu_sc as plsc


def _sc_gather_t(x, embT):
    D, V = embT.shape
    (B,) = x.shape
    info = plsc.get_sparse_core_info()
    nc = info.num_cores
    nw = nc * info.num_subcores
    bh = B * D // nw

    mesh = plsc.VectorSubcoreMesh(core_axis_name="c", subcore_axis_name="s")

    @functools.partial(
        pl.kernel,
        mesh=mesh,
        out_type=jax.ShapeDtypeStruct((D, B), jnp.float32),
        compiler_params=pltpu.CompilerParams(
            use_tc_tiling_on_sc=True, needs_layout_passes=False
        ),
        scratch_types=[
            pltpu.VMEM((V,), jnp.float32),
            pltpu.VMEM((bh,), jnp.int32),
            pltpu.VMEM((bh,), jnp.float32),
            pltpu.SemaphoreType.DMA,
            pltpu.SemaphoreType.DMA,
        ],
    )
    def gather_k(idx_hbm, table_hbm, out_hbm, row_v, idx_v, h_v, sem_r, sem_i):
        wid = lax.axis_index("s") * nc + lax.axis_index("c")
        l = wid // 2
        half = wid % 2
        cp_row = pltpu.async_copy(table_hbm.at[l], row_v, sem_r)
        cp_idx = pltpu.async_copy(idx_hbm.at[pl.ds(half * bh, bh)], idx_v, sem_i)
        cp_idx.wait()
        cp_row.wait()
        for g in range(bh // 16):
            xv = idx_v[pl.ds(g * 16, 16)]
            h_v[pl.ds(g * 16, 16)] = plsc.load_gather(row_v, [xv])
        pltpu.sync_copy(h_v, out_hbm.at[l, pl.ds(half * bh, bh)])

    return gather_k(x, embT)


def _tc_project_t(hT, Wt, brow, vt, nbuf):
    D, B = hT.shape
    V = Wt.shape[1]
    nfull = V // vt
    rem = V - nfull * vt
    grid = nfull + (1 if rem else 0)

    def body(ht_ref, wt_ref, b_ref, out_hbm, bufs, sems):
        i = pl.program_id(0)
        n = pl.num_programs(0)
        slot = lax.rem(i, nbuf)

        def copy_for(step, s, width):
            return pltpu.make_async_copy(
                bufs.at[s, pl.ds(0, width), :],
                out_hbm.at[pl.ds(step * vt, width), :],
                sems.at[s],
            )

        @pl.when(i >= nbuf)
        def _():
            copy_for(i - nbuf, slot, vt).wait()

        val = lax.dot_general(
            wt_ref[...],
            ht_ref[...],
            dimension_numbers=(((0,), (0,)), ((), ())),
            preferred_element_type=jnp.float32,
        ) + jnp.transpose(b_ref[...], (1, 0))
        bufs[slot] = val

        if rem:
            @pl.when(i < nfull)
            def _():
                copy_for(i, slot, vt).start()

            @pl.when(i == nfull)
            def _():
                copy_for(nfull, slot, rem).start()
        else:
            copy_for(i, slot, vt).start()

        @pl.when(i == n - 1)
        def _():
            for k in range(min(nbuf, grid)):
                step = grid - 1 - k
                width = rem if (rem and step == nfull) else vt
                copy_for(step, step % nbuf, width).wait()

    return pl.pallas_call(
        body,
        grid=(grid,),
        compiler_params=pltpu.CompilerParams(
            vmem_limit_bytes=100 * 1024 * 1024
        ),
        in_specs=[
            pl.BlockSpec((D, B), lambda i: (0, 0)),
            pl.BlockSpec((D, vt), lambda i: (0, i)),
            pl.BlockSpec((1, vt), lambda i: (0, i)),
        ],
        out_specs=pl.BlockSpec(memory_space=pl.ANY),
        out_shape=jax.ShapeDtypeStruct((V, B), jnp.float32),
        scratch_shapes=[
            pltpu.VMEM((nbuf, vt, B), jnp.float32),
            pltpu.SemaphoreType.DMA((nbuf,)),
        ],
    )(hT, Wt, brow)


def kernel(x, emb, W, b):
    hT = _sc_gather_t(x.astype(jnp.int32), emb.T)
    out_t = _tc_project_t(hT, W.T, b.reshape(1, -1), vt=2048, nbuf=6)
    return out_t.T


# speedup vs baseline: 1.0336x; 1.0044x over previous
import functools

import jax
import jax.numpy as jnp
from jax import lax
from jax.experimental import pallas as pl
from jax.experimental.pallas import tpu as pltpu
from jax.experimental.pallas import tpu_sc as plsc


def _sc_gather_t(x, embT):
    D, V = embT.shape
    (B,) = x.shape
    info = plsc.get_sparse_core_info()
    nc = info.num_cores
    bh = B

    mesh = plsc.VectorSubcoreMesh(core_axis_name="c", subcore_axis_name="s")

    @functools.partial(
        pl.kernel,
        mesh=mesh,
        out_type=jax.ShapeDtypeStruct((D, B), jnp.float32),
        compiler_params=pltpu.CompilerParams(
            use_tc_tiling_on_sc=True, needs_layout_passes=False
        ),
        scratch_types=[
            pltpu.VMEM((V,), jnp.float32),
            pltpu.VMEM((bh,), jnp.int32),
            pltpu.VMEM((bh,), jnp.float32),
            pltpu.SemaphoreType.DMA,
            pltpu.SemaphoreType.DMA,
        ],
    )
    def gather_k(idx_hbm, table_hbm, out_hbm, row_v, idx_v, h_v, sem_r, sem_i):
        wid = lax.axis_index("s") * nc + lax.axis_index("c")
        l = wid

        @pl.when(wid < D)
        def _():
            cp_row = pltpu.async_copy(table_hbm.at[l], row_v, sem_r)
            cp_idx = pltpu.async_copy(idx_hbm, idx_v, sem_i)
            cp_idx.wait()
            cp_row.wait()
            for g in range(bh // 16):
                xv = idx_v[pl.ds(g * 16, 16)]
                h_v[pl.ds(g * 16, 16)] = plsc.load_gather(row_v, [xv])
            pltpu.sync_copy(h_v, out_hbm.at[l])

    return gather_k(x, embT)


def _tc_project_t(hT, Wt, brow, vt, nbuf):
    D, B = hT.shape
    V = Wt.shape[1]
    nfull = V // vt
    rem = V - nfull * vt
    grid = nfull + (1 if rem else 0)

    def body(ht_ref, wt_ref, b_ref, out_hbm, bufs, sems):
        i = pl.program_id(0)
        n = pl.num_programs(0)
        slot = lax.rem(i, nbuf)

        def copy_for(step, s, width):
            return pltpu.make_async_copy(
                bufs.at[s, pl.ds(0, width), :],
                out_hbm.at[pl.ds(step * vt, width), :],
                sems.at[s],
            )

        @pl.when(i >= nbuf)
        def _():
            copy_for(i - nbuf, slot, vt).wait()

        val = lax.dot_general(
            wt_ref[...],
            ht_ref[...],
            dimension_numbers=(((0,), (0,)), ((), ())),
            preferred_element_type=jnp.float32,
        ) + jnp.transpose(b_ref[...], (1, 0))
        bufs[slot] = val

        if rem:
            @pl.when(i < nfull)
            def _():
                copy_for(i, slot, vt).start()

            @pl.when(i == nfull)
            def _():
                copy_for(nfull, slot, rem).start()
        else:
            copy_for(i, slot, vt).start()

        @pl.when(i == n - 1)
        def _():
            for k in range(min(nbuf, grid)):
                step = grid - 1 - k
                width = rem if (rem and step == nfull) else vt
                copy_for(step, step % nbuf, width).wait()

    return pl.pallas_call(
        body,
        grid=(grid,),
        compiler_params=pltpu.CompilerParams(
            vmem_limit_bytes=100 * 1024 * 1024
        ),
        in_specs=[
            pl.BlockSpec((D, B), lambda i: (0, 0)),
            pl.BlockSpec((D, vt), lambda i: (0, i)),
            pl.BlockSpec((1, vt), lambda i: (0, i)),
        ],
        out_specs=pl.BlockSpec(memory_space=pl.ANY),
        out_shape=jax.ShapeDtypeStruct((V, B), jnp.float32),
        scratch_shapes=[
            pltpu.VMEM((nbuf, vt, B), jnp.float32),
            pltpu.SemaphoreType.DMA((nbuf,)),
        ],
    )(hT, Wt, brow)


def kernel(x, emb, W, b):
    hT = _sc_gather_t(x.astype(jnp.int32), emb.T)
    out_t = _tc_project_t(hT, W.T, b.reshape(1, -1), vt=2048, nbuf=6)
    return out_t.T
